# baseline (speedup 1.0000x reference)
import jax
import jax.numpy as jnp
from jax import lax
from jax.experimental import pallas as pl
from jax.experimental.pallas import tpu as pltpu
from jax.experimental.pallas import tpu_sc as plsc

B = 16384
N = 1000
M = 200
LANES = 16
NFULL = (N // LANES) * LANES
MFULL = (M // LANES) * LANES

NC = 2
NS = 16
NW = NC * NS
ROWS_PER_W = B // NW
R = 32
G = ROWS_PER_W // R

SCALE = 1.0 / ((N - M) / N)

NBUF = 3
PRIME = 2


def _body(inp_hbm, idx_hbm, out_hbm, row_v, idx_v,
          sem_row, sem_idx, sem_out):
    wid = lax.axis_index("s") * NC + lax.axis_index("c")
    tile_base = wid * ROWS_PER_W

    scale = jnp.full((LANES,), SCALE, jnp.float32)
    zero = jnp.zeros((LANES,), jnp.float32)
    zero_i = jnp.zeros((LANES,), jnp.int32)
    iota = lax.iota(jnp.int32, LANES)
    scale_tail_mask = iota >= (LANES - (N - NFULL))
    tail_cols = (N - LANES) + iota

    def slices(g):
        b = g % NBUF
        rowbase = tile_base + g * R
        return (
            inp_hbm.at[pl.ds(rowbase, R)],
            idx_hbm.at[pl.ds(rowbase, R)],
            out_hbm.at[pl.ds(rowbase, R)],
            row_v.at[pl.ds(b * R, R)],
            idx_v.at[pl.ds(b * R, R)],
            sem_row.at[b],
            sem_idx.at[b],
            sem_out.at[b],
        )

    def start_in(g):
        inp_s, idxh_s, _, row_s, idxv_s, s_r, s_i, _ = slices(g)
        pltpu.async_copy(inp_s, row_s, s_r)
        pltpu.async_copy(idxh_s, idxv_s, s_i)

    def wait_in(g):
        inp_s, idxh_s, _, row_s, idxv_s, s_r, s_i, _ = slices(g)
        pltpu.make_async_copy(inp_s, row_s, s_r).wait()
        pltpu.make_async_copy(idxh_s, idxv_s, s_i).wait()

    def start_out(g):
        _, _, out_s, row_s, _, _, _, s_o = slices(g)
        pltpu.async_copy(row_s, out_s, s_o)

    def wait_out(g):
        _, _, out_s, row_s, _, _, _, s_o = slices(g)
        pltpu.make_async_copy(row_s, out_s, s_o).wait()

    def compute(g):
        b = g % NBUF
        rbase = b * R

        @plsc.parallel_loop(0, R, step=1, unroll=2)
        def row_body(r):
            rr = rbase + r
            rr_vec = zero_i + rr
            tv = row_v[rr, pl.ds(N - LANES, LANES)]
            plsc.store_scatter(row_v, [rr_vec, tail_cols], tv * scale,
                               mask=scale_tail_mask)
            for c in range(NFULL // LANES):
                o = c * LANES
                row_v[rr, pl.ds(o, LANES)] = (
                    row_v[rr, pl.ds(o, LANES)] * scale)
            for j in range(MFULL // LANES):
                idx = idx_v[rr, pl.ds(j * LANES, LANES)]
                plsc.store_scatter(row_v, [rr_vec, idx], zero)
            idxt = idx_v[rr, pl.ds(M - LANES, LANES)]
            plsc.store_scatter(row_v, [rr_vec, idxt], zero)

    for g in range(PRIME):
        start_in(g)

    def gbody(g, carry):
        @pl.when(g + PRIME < G)
        def _prefetch():
            @pl.when(g + PRIME - NBUF >= 0)
            def _free_buf():
                wait_out(g + PRIME - NBUF)
            start_in(g + PRIME)

        wait_in(g)
        compute(g)
        start_out(g)
        return carry

    lax.fori_loop(0, G, gbody, 0)

    for g in range(G - NBUF + PRIME, G):
        wait_out(g)


@jax.jit
def _run(inputs, mask_inds):
    mesh = plsc.VectorSubcoreMesh(core_axis_name="c", subcore_axis_name="s")
    return pl.kernel(
        _body,
        mesh=mesh,
        compiler_params=pltpu.CompilerParams(needs_layout_passes=False),
        out_type=jax.ShapeDtypeStruct((B, N), jnp.float32),
        scratch_types=[
            pltpu.VMEM((NBUF * R, N), jnp.float32),
            pltpu.VMEM((NBUF * R, M), jnp.int32),
            pltpu.SemaphoreType.DMA((NBUF,)),
            pltpu.SemaphoreType.DMA((NBUF,)),
            pltpu.SemaphoreType.DMA((NBUF,)),
        ],
    )(inputs, mask_inds)


def kernel(inputs, mask_inds):
    return _run(inputs, mask_inds.astype(jnp.int32))


# speedup vs baseline: 1.1513x; 1.1513x over previous
import jax
import jax.numpy as jnp
from jax import lax
from jax.experimental import pallas as pl
from jax.experimental.pallas import tpu as pltpu
from jax.experimental.pallas import tpu_sc as plsc

B = 16384
N = 1000
M = 200
LANES = 16
NFULL = (N // LANES) * LANES
MFULL = (M // LANES) * LANES

NC = 2
NS = 16
NW = NC * NS
ROWS_PER_W = B // NW
R = 32
G = ROWS_PER_W // R

SCALE = 1.0 / ((N - M) / N)

NBUF = 3
PRIME = 2


def _body(inp_hbm, idx_hbm, out_hbm, row_v, idx_v,
          sem_row, sem_idx, sem_out):
    wid = lax.axis_index("s") * NC + lax.axis_index("c")
    tile_base = wid * ROWS_PER_W

    scale = jnp.full((LANES,), SCALE, jnp.float32)
    zero = jnp.zeros((LANES,), jnp.float32)
    zero_i = jnp.zeros((LANES,), jnp.int32)
    iota = lax.iota(jnp.int32, LANES)
    scale_tail_mask = iota >= (LANES - (N - NFULL))
    tail_cols = (N - LANES) + iota

    def slices(g):
        b = g % NBUF
        rowbase = tile_base + g * R
        return (
            inp_hbm.at[pl.ds(rowbase, R)],
            idx_hbm.at[pl.ds(rowbase, R)],
            out_hbm.at[pl.ds(rowbase, R)],
            row_v.at[pl.ds(b * R, R)],
            idx_v.at[pl.ds(b * R, R)],
            sem_row.at[b],
            sem_idx.at[b],
            sem_out.at[b],
        )

    def start_in(g):
        inp_s, idxh_s, _, row_s, idxv_s, s_r, s_i, _ = slices(g)
        pltpu.async_copy(inp_s, row_s, s_r)
        pltpu.async_copy(idxh_s, idxv_s, s_i)

    def wait_in(g):
        inp_s, idxh_s, _, row_s, idxv_s, s_r, s_i, _ = slices(g)
        pltpu.make_async_copy(inp_s, row_s, s_r).wait()
        pltpu.make_async_copy(idxh_s, idxv_s, s_i).wait()

    def start_out(g):
        _, _, out_s, row_s, _, _, _, s_o = slices(g)
        pltpu.async_copy(row_s, out_s, s_o)

    def wait_out(g):
        _, _, out_s, row_s, _, _, _, s_o = slices(g)
        pltpu.make_async_copy(row_s, out_s, s_o).wait()

    def compute(g):
        b = g % NBUF
        rbase = b * R

        @plsc.parallel_loop(0, R, step=1, unroll=1)
        def row_body(r):
            rr = rbase + r
            rr_vec = zero_i + rr
            tv = row_v[rr, pl.ds(N - LANES, LANES)]
            plsc.store_scatter(row_v, [rr_vec, tail_cols], tv * scale,
                               mask=scale_tail_mask)
            for c in range(NFULL // LANES):
                o = c * LANES
                row_v[rr, pl.ds(o, LANES)] = (
                    row_v[rr, pl.ds(o, LANES)] * scale)
            for j in range(MFULL // LANES):
                idx = idx_v[rr, pl.ds(j * LANES, LANES)]
                plsc.store_scatter(row_v, [rr_vec, idx], zero)
            idxt = idx_v[rr, pl.ds(M - LANES, LANES)]
            plsc.store_scatter(row_v, [rr_vec, idxt], zero)

    for g in range(PRIME):
        start_in(g)

    def gbody(g, carry):
        @pl.when(g + PRIME < G)
        def _prefetch():
            @pl.when(g + PRIME - NBUF >= 0)
            def _free_buf():
                wait_out(g + PRIME - NBUF)
            start_in(g + PRIME)

        wait_in(g)
        compute(g)
        start_out(g)
        return carry

    lax.fori_loop(0, G, gbody, 0)

    for g in range(G - NBUF, G):
        wait_out(g)


@jax.jit
def _run(inputs, mask_inds):
    mesh = plsc.VectorSubcoreMesh(core_axis_name="c", subcore_axis_name="s")
    return pl.kernel(
        _body,
        mesh=mesh,
        compiler_params=pltpu.CompilerParams(needs_layout_passes=False),
        out_type=jax.ShapeDtypeStruct((B, N), jnp.float32),
        scratch_types=[
            pltpu.VMEM((NBUF * R, N), jnp.float32),
            pltpu.VMEM((NBUF * R, M), jnp.int32),
            pltpu.SemaphoreType.DMA((NBUF,)),
            pltpu.SemaphoreType.DMA((NBUF,)),
            pltpu.SemaphoreType.DMA((NBUF,)),
        ],
    )(inputs, mask_inds)


def kernel(inputs, mask_inds):
    return _run(inputs, mask_inds.astype(jnp.int32))
